# baseline (speedup 1.0000x reference)
import functools

import jax
import jax.numpy as jnp
from jax import lax
from jax.experimental import pallas as pl
from jax.experimental.pallas import tpu as pltpu
from jax.experimental.pallas import tpu_sc as plsc

_SCORE_THRESHOLD = 0.05
_IOU_THRESHOLD = 0.5
_MAX_DET = 300
_NEG = float("-inf")

_B = 8
_N = 5000
_C = 80
_NPAD = 5008
_NCHUNK = _NPAD // 16
_CMPAD = 320
_OUTPAD = 304


def _prep_body(cls_ref, s_ref, lab_ref):
    def cbody(c, carry):
        best, labv = carry
        v = cls_ref[c]
        upd = v > best
        return jnp.where(upd, v, best), jnp.where(upd, c, labv)

    best0 = cls_ref[0]
    lab0 = jnp.zeros((_B, _N), jnp.int32)
    best, labv = lax.fori_loop(1, _C, cbody, (best0, lab0))
    s_ref[:, :_N] = jnp.where(best > _SCORE_THRESHOLD, best, _NEG)
    s_ref[:, _N:] = jnp.full((_B, _NPAD - _N), _NEG, jnp.float32)
    lab_ref[:, :_N] = labv
    lab_ref[:, _N:] = jnp.zeros((_B, _NPAD - _N), jnp.int32)


def _sc_nms(s_hbm, lab_hbm, box_hbm,
            oy1_hbm, ox1_hbm, oy2_hbm, ox2_hbm, osc_hbm, olab_hbm,
            s_v, lab_v, box_v, cm_v, gm_v,
            ky1_v, kx1_v, ky2_v, kx2_v, karea_v, osc_v, olab_v):
    w = lax.axis_index("s") * 2 + lax.axis_index("c")

    @pl.when(w < _B)
    def _():
        b = w
        pltpu.sync_copy(s_hbm.at[b], s_v)
        pltpu.sync_copy(lab_hbm.at[b], lab_v.at[pl.ds(0, _NPAD)])
        pltpu.sync_copy(box_hbm.at[b], box_v.at[pl.ds(0, _N * 4)])

        iota = lax.iota(jnp.int32, 16)
        negv = jnp.full((16,), _NEG, jnp.float32)
        m1f = jnp.full((16,), -1.0, jnp.float32)
        m1i = jnp.full((16,), -1, jnp.int32)
        z16 = jnp.zeros((16,), jnp.float32)

        for k in range(_CMPAD // 16):
            cm_v[pl.ds(16 * k, 16)] = negv
        for k in range(_OUTPAD // 16):
            sl = pl.ds(16 * k, 16)
            ky1_v[sl] = m1f
            kx1_v[sl] = m1f
            ky2_v[sl] = m1f
            kx2_v[sl] = m1f
            karea_v[sl] = z16
            osc_v[sl] = m1f
            olab_v[sl] = m1i

        def cmbody(k, _):
            m = jnp.max(s_v[pl.ds(k * 16, 16)])
            base = (k // 16) * 16
            lane = k - base
            old = cm_v[pl.ds(base, 16)]
            cm_v[pl.ds(base, 16)] = jnp.where(iota == lane,
                                              jnp.full((16,), m, jnp.float32),
                                              old)
            return 0
        lax.fori_loop(0, _NCHUNK, cmbody, 0)

        gm_v[pl.ds(0, 16)] = negv
        gm_v[pl.ds(16, 16)] = negv
        for gk in range(_CMPAD // 256 + 1):
            base = gk * 16
            ngrp = min(16, _CMPAD // 16 - base)
            acc = gm_v[pl.ds(base, 16)]
            for j in range(ngrp):
                m = jnp.max(cm_v[pl.ds((base + j) * 16, 16)])
                acc = jnp.where(iota == j, jnp.full((16,), m, jnp.float32), acc)
            gm_v[pl.ds(base, 16)] = acc

        BIG = jnp.int32(10 ** 6)

        def cond(st):
            cnt, done = st
            return jnp.logical_and(cnt < _MAX_DET, jnp.logical_not(done))

        def body(st):
            cnt, done = st
            g0 = gm_v[pl.ds(0, 16)]
            g1 = gm_v[pl.ds(16, 16)]
            best = jnp.maximum(jnp.max(g0), jnp.max(g1))
            valid = best > _NEG
            bestv = jnp.full((16,), best, jnp.float32)
            i0 = jnp.min(jnp.where(g0 == bestv, iota, BIG))
            i1 = jnp.min(jnp.where(g1 == bestv, iota + 16, BIG))
            gidx = jnp.minimum(i0, i1)
            cmv = cm_v[pl.ds(gidx * 16, 16)]
            clane = jnp.min(jnp.where(cmv == bestv, iota, BIG))
            cstar = gidx * 16 + clane
            sv = s_v[pl.ds(cstar * 16, 16)]
            lanew = jnp.min(jnp.where(sv == bestv, iota, BIG))
            g = cstar * 16 + lanew
            gc = jnp.minimum(g, _N - 1)

            bv = box_v[pl.ds(gc * 4, 16)]
            cy1s = bv[0]
            cx1s = bv[1]
            cy2s = bv[2]
            cx2s = bv[3]
            clabs = lab_v[pl.ds(gc, 16)][0]
            cy1 = jnp.full((16,), cy1s, jnp.float32)
            cx1 = jnp.full((16,), cx1s, jnp.float32)
            cy2 = jnp.full((16,), cy2s, jnp.float32)
            cx2 = jnp.full((16,), cx2s, jnp.float32)
            careas = (cy2s - cy1s) * (cx2s - cx1s)
            carea = jnp.full((16,), careas, jnp.float32)

            newsv = jnp.where(iota == lanew, negv, sv)
            s_v[pl.ds(cstar * 16, 16)] = newsv
            newm = jnp.max(newsv)
            newcm = jnp.where(iota == clane,
                              jnp.full((16,), newm, jnp.float32), cmv)
            cm_v[pl.ds(gidx * 16, 16)] = newcm
            newgm = jnp.max(newcm)
            gbase = (gidx // 16) * 16
            glane = gidx - gbase
            oldgm = gm_v[pl.ds(gbase, 16)]
            gm_v[pl.ds(gbase, 16)] = jnp.where(
                iota == glane, jnp.full((16,), newgm, jnp.float32), oldgm)

            nk = (cnt + 15) // 16

            def jbody(j, suppacc):
                sl = pl.ds(j * 16, 16)
                ky1 = ky1_v[sl]
                kx1 = kx1_v[sl]
                ky2 = ky2_v[sl]
                kx2 = kx2_v[sl]
                karea = karea_v[sl]
                yy1 = jnp.maximum(ky1, cy1)
                xx1 = jnp.maximum(kx1, cx1)
                yy2 = jnp.minimum(ky2, cy2)
                xx2 = jnp.minimum(kx2, cx2)
                inter = jnp.maximum(0.0, yy2 - yy1) * jnp.maximum(0.0, xx2 - xx1)
                union = karea + carea - inter
                iou = jnp.where(union > 0, inter / union, 0.0)
                return jnp.logical_or(suppacc, iou > _IOU_THRESHOLD)

            suppv = lax.fori_loop(0, nk, jbody, jnp.zeros((16,), jnp.bool_))
            anysupp = jnp.max(jnp.where(suppv, 1, 0).astype(jnp.int32)) > 0
            accept = jnp.logical_and(valid, jnp.logical_not(anysupp))

            obase = (cnt // 16) * 16
            olane = cnt - obase
            am = jnp.logical_and(iota == olane, jnp.full((16,), accept))
            osl = pl.ds(obase, 16)

            def blend_f(ref, vals):
                ref[osl] = jnp.where(am, jnp.full((16,), vals, jnp.float32),
                                     ref[osl])

            blend_f(ky1_v, cy1s)
            blend_f(kx1_v, cx1s)
            blend_f(ky2_v, cy2s)
            blend_f(kx2_v, cx2s)
            blend_f(karea_v, careas)
            blend_f(osc_v, best)
            olab_v[osl] = jnp.where(am, jnp.full((16,), clabs - 1, jnp.int32),
                                    olab_v[osl])

            cnt2 = cnt + jnp.where(accept, 1, 0).astype(jnp.int32)
            return cnt2, jnp.logical_not(valid)

        lax.while_loop(cond, body, (jnp.int32(0), jnp.bool_(False)))

        pltpu.sync_copy(ky1_v, oy1_hbm.at[b])
        pltpu.sync_copy(kx1_v, ox1_hbm.at[b])
        pltpu.sync_copy(ky2_v, oy2_hbm.at[b])
        pltpu.sync_copy(kx2_v, ox2_hbm.at[b])
        pltpu.sync_copy(osc_v, osc_hbm.at[b])
        pltpu.sync_copy(olab_v, olab_hbm.at[b])


_sc_nms_call = functools.partial(
    pl.kernel,
    out_type=(
        jax.ShapeDtypeStruct((_B, _OUTPAD), jnp.float32),
        jax.ShapeDtypeStruct((_B, _OUTPAD), jnp.float32),
        jax.ShapeDtypeStruct((_B, _OUTPAD), jnp.float32),
        jax.ShapeDtypeStruct((_B, _OUTPAD), jnp.float32),
        jax.ShapeDtypeStruct((_B, _OUTPAD), jnp.float32),
        jax.ShapeDtypeStruct((_B, _OUTPAD), jnp.int32),
    ),
    mesh=plsc.VectorSubcoreMesh(core_axis_name="c", subcore_axis_name="s"),
    compiler_params=pltpu.CompilerParams(needs_layout_passes=False,
                                         use_tc_tiling_on_sc=False),
    scratch_types=[
        pltpu.VMEM((_NPAD,), jnp.float32),
        pltpu.VMEM((_NPAD + 16,), jnp.int32),
        pltpu.VMEM((_N * 4 + 16,), jnp.float32),
        pltpu.VMEM((_CMPAD,), jnp.float32),
        pltpu.VMEM((32,), jnp.float32),
        pltpu.VMEM((_OUTPAD,), jnp.float32),
        pltpu.VMEM((_OUTPAD,), jnp.float32),
        pltpu.VMEM((_OUTPAD,), jnp.float32),
        pltpu.VMEM((_OUTPAD,), jnp.float32),
        pltpu.VMEM((_OUTPAD,), jnp.float32),
        pltpu.VMEM((_OUTPAD,), jnp.float32),
        pltpu.VMEM((_OUTPAD,), jnp.int32),
    ],
)(_sc_nms)


def kernel(boxes, classification):
    cls_t = jnp.transpose(classification, (2, 0, 1))
    s_pad, lab_pad = pl.pallas_call(
        _prep_body,
        out_shape=(
            jax.ShapeDtypeStruct((_B, _NPAD), jnp.float32),
            jax.ShapeDtypeStruct((_B, _NPAD), jnp.int32),
        ),
    )(cls_t)
    box_flat = boxes.reshape(_B, _N * 4)
    oy1, ox1, oy2, ox2, osc, olab = _sc_nms_call(s_pad, lab_pad, box_flat)
    out_boxes = jnp.stack(
        [oy1[:, :_MAX_DET], ox1[:, :_MAX_DET],
         oy2[:, :_MAX_DET], ox2[:, :_MAX_DET]], axis=-1)
    return out_boxes, osc[:, :_MAX_DET], olab[:, :_MAX_DET]


# speedup vs baseline: 1.0394x; 1.0394x over previous
import functools

import jax
import jax.numpy as jnp
from jax import lax
from jax.experimental import pallas as pl
from jax.experimental.pallas import tpu as pltpu
from jax.experimental.pallas import tpu_sc as plsc

_SCORE_THRESHOLD = 0.05
_IOU_THRESHOLD = 0.5
_MAX_DET = 300
_NEG = float("-inf")

_B = 8
_N = 5000
_C = 80
_NPAD = 5008
_NCHUNK = _NPAD // 16
_CMPAD = 320
_OUTPAD = 304


def _prep_body(cls_ref, s_ref, lab_ref):
    def cbody(c, carry):
        best, labv = carry
        v = cls_ref[c]
        upd = v > best
        return jnp.where(upd, v, best), jnp.where(upd, c, labv)

    best0 = cls_ref[0]
    lab0 = jnp.zeros((_B, _N), jnp.int32)
    best, labv = lax.fori_loop(1, _C, cbody, (best0, lab0))
    s_ref[:, :_N] = jnp.where(best > _SCORE_THRESHOLD, best, _NEG)
    s_ref[:, _N:] = jnp.full((_B, _NPAD - _N), _NEG, jnp.float32)
    lab_ref[:, :_N] = labv
    lab_ref[:, _N:] = jnp.zeros((_B, _NPAD - _N), jnp.int32)


def _sc_nms(s_hbm, lab_hbm, box_hbm,
            oy1_hbm, ox1_hbm, oy2_hbm, ox2_hbm, osc_hbm, olab_hbm,
            s_v, lab_v, box_v, cm_v,
            ky1_v, kx1_v, ky2_v, kx2_v, karea_v, osc_v, olab_v):
    w = lax.axis_index("s") * 2 + lax.axis_index("c")

    @pl.when(w < _B)
    def _():
        b = w
        pltpu.sync_copy(s_hbm.at[b], s_v)
        pltpu.sync_copy(lab_hbm.at[b], lab_v.at[pl.ds(0, _NPAD)])
        pltpu.sync_copy(box_hbm.at[b], box_v.at[pl.ds(0, _N * 4)])

        iota = lax.iota(jnp.int32, 16)
        negv = jnp.full((16,), _NEG, jnp.float32)
        m1f = jnp.full((16,), -1.0, jnp.float32)
        m1i = jnp.full((16,), -1, jnp.int32)
        z16 = jnp.zeros((16,), jnp.float32)

        for k in range(_CMPAD // 16):
            cm_v[pl.ds(16 * k, 16)] = negv
        for k in range(_OUTPAD // 16):
            sl = pl.ds(16 * k, 16)
            ky1_v[sl] = m1f
            kx1_v[sl] = m1f
            ky2_v[sl] = m1f
            kx2_v[sl] = m1f
            karea_v[sl] = z16
            osc_v[sl] = m1f
            olab_v[sl] = m1i

        def cmbody(k, _):
            m = jnp.max(s_v[pl.ds(k * 16, 16)])
            base = (k // 16) * 16
            lane = k - base
            old = cm_v[pl.ds(base, 16)]
            cm_v[pl.ds(base, 16)] = jnp.where(iota == lane,
                                              jnp.full((16,), m, jnp.float32),
                                              old)
            return 0
        lax.fori_loop(0, _NCHUNK, cmbody, 0)

        BIG = jnp.int32(10 ** 6)
        _NREG = _CMPAD // 16

        def _tree(op, vs):
            vs = list(vs)
            while len(vs) > 1:
                vs = [op(vs[i], vs[i + 1]) if i + 1 < len(vs) else vs[i]
                      for i in range(0, len(vs), 2)]
            return vs[0]

        cms0 = tuple(cm_v[pl.ds(16 * k, 16)] for k in range(_NREG))

        def cond(st):
            cnt, done = st[0], st[1]
            return jnp.logical_and(cnt < _MAX_DET, jnp.logical_not(done))

        def body(st):
            cnt, done, cms = st[0], st[1], st[2]
            best = jnp.max(_tree(jnp.maximum, cms))
            valid = best > _NEG
            bestv = jnp.full((16,), best, jnp.float32)
            idxs = [jnp.where(cms[k] == bestv, iota + 16 * k, BIG)
                    for k in range(_NREG)]
            cstar = jnp.min(_tree(jnp.minimum, idxs))
            sv = s_v[pl.ds(cstar * 16, 16)]
            lanew = jnp.min(jnp.where(sv == bestv, iota, BIG))
            g = cstar * 16 + lanew
            gc = jnp.minimum(g, _N - 1)

            bv = box_v[pl.ds(gc * 4, 16)]
            cy1s = bv[0]
            cx1s = bv[1]
            cy2s = bv[2]
            cx2s = bv[3]
            clabs = lab_v[pl.ds(gc, 16)][0]
            cy1 = jnp.full((16,), cy1s, jnp.float32)
            cx1 = jnp.full((16,), cx1s, jnp.float32)
            cy2 = jnp.full((16,), cy2s, jnp.float32)
            cx2 = jnp.full((16,), cx2s, jnp.float32)
            careas = (cy2s - cy1s) * (cx2s - cx1s)
            carea = jnp.full((16,), careas, jnp.float32)

            newsv = jnp.where(iota == lanew, negv, sv)
            s_v[pl.ds(cstar * 16, 16)] = newsv
            newm = jnp.max(newsv)
            newmv = jnp.full((16,), newm, jnp.float32)
            kreg = cstar // 16
            klane = cstar - kreg * 16
            cms_new = tuple(
                jnp.where(kreg == k, jnp.where(iota == klane, newmv, cms[k]),
                          cms[k])
                for k in range(_NREG))

            nk = (cnt + 15) // 16

            def jbody(j, suppacc):
                sl = pl.ds(j * 16, 16)
                ky1 = ky1_v[sl]
                kx1 = kx1_v[sl]
                ky2 = ky2_v[sl]
                kx2 = kx2_v[sl]
                karea = karea_v[sl]
                yy1 = jnp.maximum(ky1, cy1)
                xx1 = jnp.maximum(kx1, cx1)
                yy2 = jnp.minimum(ky2, cy2)
                xx2 = jnp.minimum(kx2, cx2)
                inter = jnp.maximum(0.0, yy2 - yy1) * jnp.maximum(0.0, xx2 - xx1)
                union = karea + carea - inter
                iou = jnp.where(union > 0, inter / union, 0.0)
                return jnp.logical_or(suppacc, iou > _IOU_THRESHOLD)

            suppv = lax.fori_loop(0, nk, jbody, jnp.zeros((16,), jnp.bool_))
            anysupp = jnp.max(jnp.where(suppv, 1, 0).astype(jnp.int32)) > 0
            accept = jnp.logical_and(valid, jnp.logical_not(anysupp))

            obase = (cnt // 16) * 16
            olane = cnt - obase
            am = jnp.logical_and(iota == olane, jnp.full((16,), accept))
            osl = pl.ds(obase, 16)

            def blend_f(ref, vals):
                ref[osl] = jnp.where(am, jnp.full((16,), vals, jnp.float32),
                                     ref[osl])

            blend_f(ky1_v, cy1s)
            blend_f(kx1_v, cx1s)
            blend_f(ky2_v, cy2s)
            blend_f(kx2_v, cx2s)
            blend_f(karea_v, careas)
            blend_f(osc_v, best)
            olab_v[osl] = jnp.where(am, jnp.full((16,), clabs - 1, jnp.int32),
                                    olab_v[osl])

            cnt2 = cnt + jnp.where(accept, 1, 0).astype(jnp.int32)
            return cnt2, jnp.logical_not(valid), cms_new

        lax.while_loop(cond, body, (jnp.int32(0), jnp.bool_(False), cms0))

        pltpu.sync_copy(ky1_v, oy1_hbm.at[b])
        pltpu.sync_copy(kx1_v, ox1_hbm.at[b])
        pltpu.sync_copy(ky2_v, oy2_hbm.at[b])
        pltpu.sync_copy(kx2_v, ox2_hbm.at[b])
        pltpu.sync_copy(osc_v, osc_hbm.at[b])
        pltpu.sync_copy(olab_v, olab_hbm.at[b])


_sc_nms_call = functools.partial(
    pl.kernel,
    out_type=(
        jax.ShapeDtypeStruct((_B, _OUTPAD), jnp.float32),
        jax.ShapeDtypeStruct((_B, _OUTPAD), jnp.float32),
        jax.ShapeDtypeStruct((_B, _OUTPAD), jnp.float32),
        jax.ShapeDtypeStruct((_B, _OUTPAD), jnp.float32),
        jax.ShapeDtypeStruct((_B, _OUTPAD), jnp.float32),
        jax.ShapeDtypeStruct((_B, _OUTPAD), jnp.int32),
    ),
    mesh=plsc.VectorSubcoreMesh(core_axis_name="c", subcore_axis_name="s"),
    compiler_params=pltpu.CompilerParams(needs_layout_passes=False,
                                         use_tc_tiling_on_sc=False),
    scratch_types=[
        pltpu.VMEM((_NPAD,), jnp.float32),
        pltpu.VMEM((_NPAD + 16,), jnp.int32),
        pltpu.VMEM((_N * 4 + 16,), jnp.float32),
        pltpu.VMEM((_CMPAD,), jnp.float32),
        pltpu.VMEM((_OUTPAD,), jnp.float32),
        pltpu.VMEM((_OUTPAD,), jnp.float32),
        pltpu.VMEM((_OUTPAD,), jnp.float32),
        pltpu.VMEM((_OUTPAD,), jnp.float32),
        pltpu.VMEM((_OUTPAD,), jnp.float32),
        pltpu.VMEM((_OUTPAD,), jnp.float32),
        pltpu.VMEM((_OUTPAD,), jnp.int32),
    ],
)(_sc_nms)


def kernel(boxes, classification):
    cls_t = jnp.transpose(classification, (2, 0, 1))
    s_pad, lab_pad = pl.pallas_call(
        _prep_body,
        out_shape=(
            jax.ShapeDtypeStruct((_B, _NPAD), jnp.float32),
            jax.ShapeDtypeStruct((_B, _NPAD), jnp.int32),
        ),
    )(cls_t)
    box_flat = boxes.reshape(_B, _N * 4)
    oy1, ox1, oy2, ox2, osc, olab = _sc_nms_call(s_pad, lab_pad, box_flat)
    out_boxes = jnp.stack(
        [oy1[:, :_MAX_DET], ox1[:, :_MAX_DET],
         oy2[:, :_MAX_DET], ox2[:, :_MAX_DET]], axis=-1)
    return out_boxes, osc[:, :_MAX_DET], olab[:, :_MAX_DET]
